# Initial kernel scaffold; baseline (speedup 1.0000x reference)
import functools

import jax
import jax.numpy as jnp
from jax import lax
from jax.experimental import pallas as pl
from jax.experimental.pallas import tpu as pltpu
from jax.experimental.pallas import tpu_sc as plsc

_NC = 2
_NS = 16
_CHUNK = 128


def _ceil_mult(v, m):
    return (v + m - 1) // m * m


def _sc_degree(dstp, npad, ncw):
    mesh = plsc.VectorSubcoreMesh(core_axis_name="c", subcore_axis_name="s")
    rows_sub = npad // _NS
    ew = ncw * _CHUNK

    @functools.partial(
        pl.kernel,
        out_type=jax.ShapeDtypeStruct((_NC, npad, 16), jnp.float32),
        mesh=mesh,
        scratch_types=[
            pltpu.VMEM((_CHUNK,), jnp.int32),
            pltpu.VMEM((_CHUNK, 16), jnp.float32),
            pltpu.VMEM((_CHUNK, 16), jnp.float32),
            pltpu.VMEM_SHARED((npad, 16), jnp.float32),
            pltpu.SemaphoreType.DMA,
        ],
    )
    def k(dst_hbm, out_hbm, didx, ones, ztile, acc, sem):
        cid = lax.axis_index("c")
        sid = lax.axis_index("s")

        @pl.loop(0, _CHUNK)
        def _(r):
            ones[r, :] = jnp.ones((16,), jnp.float32)
            ztile[r, :] = jnp.zeros((16,), jnp.float32)

        base_r = sid * rows_sub

        @pl.loop(0, rows_sub, step=_CHUNK)
        def _(r):
            pltpu.sync_copy(ztile, acc.at[pl.ds(base_r + r, _CHUNK)])

        plsc.subcore_barrier()

        wid = sid * _NC + cid
        ebase = wid * ew

        @pl.loop(0, ncw)
        def _(c):
            pltpu.sync_copy(dst_hbm.at[pl.ds(ebase + c * _CHUNK, _CHUNK)], didx)
            pltpu.sync_copy(ones, acc.at[didx], add=True)

        plsc.subcore_barrier()
        pltpu.sync_copy(acc.at[pl.ds(base_r, rows_sub)],
                        out_hbm.at[cid, pl.ds(base_r, rows_sub)])

    return k(dstp)


def _sc_spmm(y, srcp, dstp, npad, ncw):
    d = y.shape[1]
    mesh = plsc.VectorSubcoreMesh(core_axis_name="c", subcore_axis_name="s")
    rows_sub = npad // _NS
    ew = ncw * _CHUNK
    zrows = 16

    @functools.partial(
        pl.kernel,
        out_type=jax.ShapeDtypeStruct((_NC, npad, d), jnp.float32),
        mesh=mesh,
        scratch_types=[
            pltpu.VMEM((_CHUNK,), jnp.int32),
            pltpu.VMEM((_CHUNK,), jnp.int32),
            pltpu.VMEM((_CHUNK, d), jnp.float32),
            pltpu.VMEM((zrows, d), jnp.float32),
            pltpu.VMEM_SHARED((npad, d), jnp.float32),
            pltpu.SemaphoreType.DMA,
        ],
    )
    def k(y_hbm, src_hbm, dst_hbm, out_hbm, sidx, didx, gbuf, ztile, acc, sem):
        cid = lax.axis_index("c")
        sid = lax.axis_index("s")

        @pl.loop(0, zrows)
        def _(r):
            @pl.loop(0, d, step=16)
            def _(j):
                ztile[r, pl.ds(j, 16)] = jnp.zeros((16,), jnp.float32)

        base_r = sid * rows_sub

        @pl.loop(0, rows_sub, step=zrows)
        def _(r):
            pltpu.sync_copy(ztile, acc.at[pl.ds(base_r + r, zrows)])

        plsc.subcore_barrier()

        wid = sid * _NC + cid
        ebase = wid * ew

        @pl.loop(0, ncw)
        def _(c):
            off = ebase + c * _CHUNK
            pltpu.sync_copy(src_hbm.at[pl.ds(off, _CHUNK)], sidx)
            pltpu.sync_copy(dst_hbm.at[pl.ds(off, _CHUNK)], didx)
            pltpu.async_copy(y_hbm.at[sidx], gbuf, sem).wait()
            pltpu.sync_copy(gbuf, acc.at[didx], add=True)

        plsc.subcore_barrier()
        pltpu.sync_copy(acc.at[pl.ds(base_r, rows_sub)],
                        out_hbm.at[cid, pl.ds(base_r, rows_sub)])

    return k(y, srcp, dstp)


def _dis_from_dacc(d_ref, n, ypad):
    deg = d_ref[0, 0:ypad, 0:1] + d_ref[1, 0:ypad, 0:1] + 1.0
    row = lax.broadcasted_iota(jnp.int32, (ypad, 1), 0)
    return jnp.where(row < n, lax.rsqrt(deg), 0.0)


def _tc_stage1(xpad, w1, dacc, n):
    ypad, d_in = xpad.shape
    d_hid = w1.shape[1]

    def body(x_ref, w_ref, d_ref, y_ref):
        dis = _dis_from_dacc(d_ref, n, ypad)
        xw = jnp.dot(x_ref[...], w_ref[...],
                     preferred_element_type=jnp.float32,
                     precision=lax.Precision.HIGHEST)
        y_ref[...] = xw * dis

    return pl.pallas_call(
        body,
        out_shape=jax.ShapeDtypeStruct((ypad, d_hid), jnp.float32),
    )(xpad, w1, dacc)


def _tc_stage2(dacc, a1, y1, b1, w2a, w2b, xroot, n):
    ypad, d_hid = y1.shape
    d_out = w2a.shape[1]

    def body(d_ref, a_ref, y_ref, b_ref, wa_ref, wb_ref, xr_ref,
             y2_ref, c1_ref):
        dis = _dis_from_dacc(d_ref, n, ypad)
        s = a_ref[0, 0:ypad, :] + a_ref[1, 0:ypad, :] + y_ref[...]
        conv1 = dis * s + b_ref[...]
        h = jnp.maximum(conv1, 0.0)
        crow = jnp.dot(jnp.maximum(xr_ref[...], 0.0), wb_ref[...],
                       preferred_element_type=jnp.float32,
                       precision=lax.Precision.HIGHEST)
        xw2 = jnp.dot(h, wa_ref[...],
                      preferred_element_type=jnp.float32,
                      precision=lax.Precision.HIGHEST) + crow
        y2_ref[...] = xw2 * dis
        c1_ref[...] = conv1

    return pl.pallas_call(
        body,
        out_shape=(
            jax.ShapeDtypeStruct((ypad, d_out), jnp.float32),
            jax.ShapeDtypeStruct((ypad, d_hid), jnp.float32),
        ),
    )(dacc, a1, y1, b1, w2a, w2b, xroot)


def _tc_stage3(dacc, a2, y2, b2, c1root, n):
    ypad, d_out = y2.shape
    d_hid = c1root.shape[1]
    d_feat = d_hid + d_out

    def body(d_ref, a_ref, y_ref, b_ref, r_ref, o_ref):
        dis = _dis_from_dacc(d_ref, n, ypad)
        s = a_ref[0, 0:ypad, :] + a_ref[1, 0:ypad, :] + y_ref[...]
        conv2 = dis * s + b_ref[...]
        r2 = jnp.maximum(conv2, 0.0)[0:n, :]
        f = jnp.concatenate(
            [jnp.broadcast_to(r_ref[...], (n, d_hid)), r2], axis=1)
        o_ref[...] = (f[:, 0:d_feat - 2] + f[:, 1:d_feat - 1]
                      + f[:, 2:d_feat]) * (1.0 / 3.0)

    return pl.pallas_call(
        body,
        out_shape=jax.ShapeDtypeStruct((n, d_feat - 2), jnp.float32),
    )(dacc, a2, y2, b2, c1root)


def kernel(x, edge_index, rootIndex, W1, b1, W2, b2):
    n, d_in = x.shape
    d_hid = W1.shape[1]
    e = edge_index.shape[1]
    nw = _NC * _NS

    ypad = n + 16
    npad = _ceil_mult(n + 1, _NS * 16)
    epad = _ceil_mult(e, nw * _CHUNK)
    ncw = epad // (nw * _CHUNK)

    src = edge_index[0]
    dst = edge_index[1]
    pad = epad - e
    fill = jnp.full((pad,), n, jnp.int32)
    srcp = jnp.concatenate([src, fill])
    dstp = jnp.concatenate([dst, fill])
    xpad = jnp.concatenate(
        [x, jnp.zeros((ypad - n, d_in), x.dtype)], axis=0)

    dacc = _sc_degree(dstp, npad, ncw)
    y1 = _tc_stage1(xpad, W1, dacc, n)
    a1 = _sc_spmm(y1, srcp, dstp, npad, ncw)
    xroot = lax.dynamic_slice_in_dim(x, rootIndex, 1, axis=0)
    y2, conv1 = _tc_stage2(dacc, a1, y1, b1.reshape(1, -1),
                           W2[:d_hid], W2[d_hid:], xroot, n)
    a2 = _sc_spmm(y2, srcp, dstp, npad, ncw)
    c1root = lax.dynamic_slice_in_dim(conv1, rootIndex, 1, axis=0)
    return _tc_stage3(dacc, a2, y2, b2.reshape(1, -1), c1root, n)


# speedup vs baseline: 12.0285x; 12.0285x over previous
import functools

import jax
import jax.numpy as jnp
from jax import lax
from jax.experimental import pallas as pl
from jax.experimental.pallas import tpu as pltpu
from jax.experimental.pallas import tpu_sc as plsc

_NC = 2
_NS = 16
_CHUNK = 128
_RB = 1280


def _ceil_mult(v, m):
    return (v + m - 1) // m * m


def _mesh():
    return plsc.VectorSubcoreMesh(core_axis_name="c", subcore_axis_name="s",
                                  num_cores=_NC, num_subcores=_NS)


def _sc_degree(dstp, npad, ncw):
    rows_sub = npad // _NS
    ew = ncw * _CHUNK

    @functools.partial(
        pl.kernel,
        out_type=jax.ShapeDtypeStruct((_NC, npad, 16), jnp.float32),
        mesh=_mesh(),
        scratch_types=[
            pltpu.VMEM((_CHUNK,), jnp.int32),
            pltpu.VMEM((_CHUNK, 16), jnp.float32),
            pltpu.VMEM((_CHUNK, 16), jnp.float32),
            pltpu.VMEM_SHARED((npad, 16), jnp.float32),
            pltpu.SemaphoreType.DMA,
        ],
    )
    def k(dst_hbm, out_hbm, didx, ones, ztile, acc, sem):
        cid = lax.axis_index("c")
        sid = lax.axis_index("s")

        @pl.loop(0, _CHUNK)
        def _(r):
            ones[r, :] = jnp.ones((16,), jnp.float32)
            ztile[r, :] = jnp.zeros((16,), jnp.float32)

        base_r = sid * rows_sub

        @pl.loop(0, rows_sub, step=_CHUNK)
        def _(r):
            pltpu.sync_copy(ztile, acc.at[pl.ds(base_r + r, _CHUNK)])

        plsc.subcore_barrier()

        wid = sid * _NC + cid
        ebase = wid * ew

        @pl.loop(0, ncw)
        def _(c):
            pltpu.sync_copy(dst_hbm.at[pl.ds(ebase + c * _CHUNK, _CHUNK)], didx)
            pltpu.sync_copy(ones, acc.at[didx], add=True)

        plsc.subcore_barrier()
        pltpu.sync_copy(acc.at[pl.ds(base_r, rows_sub)],
                        out_hbm.at[cid, pl.ds(base_r, rows_sub)])

    return k(dstp)


def _sc_spmm(y, srcp, dstp, npad, ncw):
    d = y.shape[1]
    rows_sub = npad // _NS
    ew = ncw * _CHUNK
    zrows = 16

    @functools.partial(
        pl.kernel,
        out_type=jax.ShapeDtypeStruct((_NC, npad, d), jnp.float32),
        mesh=_mesh(),
        scratch_types=[
            pltpu.VMEM((_CHUNK,), jnp.int32),
            pltpu.VMEM((_CHUNK,), jnp.int32),
            pltpu.VMEM((_CHUNK, d), jnp.float32),
            pltpu.VMEM((zrows, d), jnp.float32),
            pltpu.VMEM_SHARED((npad, d), jnp.float32),
            pltpu.SemaphoreType.DMA,
        ],
    )
    def k(y_hbm, src_hbm, dst_hbm, out_hbm, sidx, didx, gbuf, ztile, acc, sem):
        cid = lax.axis_index("c")
        sid = lax.axis_index("s")

        @pl.loop(0, zrows)
        def _(r):
            @pl.loop(0, d, step=16)
            def _(j):
                ztile[r, pl.ds(j, 16)] = jnp.zeros((16,), jnp.float32)

        base_r = sid * rows_sub

        @pl.loop(0, rows_sub, step=zrows)
        def _(r):
            pltpu.sync_copy(ztile, acc.at[pl.ds(base_r + r, zrows)])

        plsc.subcore_barrier()

        wid = sid * _NC + cid
        ebase = wid * ew

        @pl.loop(0, ncw)
        def _(c):
            off = ebase + c * _CHUNK
            pltpu.sync_copy(src_hbm.at[pl.ds(off, _CHUNK)], sidx)
            pltpu.sync_copy(dst_hbm.at[pl.ds(off, _CHUNK)], didx)
            pltpu.async_copy(y_hbm.at[sidx], gbuf, sem).wait()
            pltpu.sync_copy(gbuf, acc.at[didx], add=True)

        plsc.subcore_barrier()
        pltpu.sync_copy(acc.at[pl.ds(base_r, rows_sub)],
                        out_hbm.at[cid, pl.ds(base_r, rows_sub)])

    return k(y, srcp, dstp)


def _dis_block(d_ref, n):
    deg = d_ref[0, :, 0:1] + d_ref[1, :, 0:1] + 1.0
    row = (pl.program_id(0) * _RB
           + lax.broadcasted_iota(jnp.int32, (_RB, 1), 0))
    return jnp.where(row < n, lax.rsqrt(deg), 0.0)


def _dspec():
    return pl.BlockSpec((2, _RB, 16), lambda i: (0, i, 0))


def _full(shape):
    nd = len(shape)
    return pl.BlockSpec(shape, lambda i: (0,) * nd)


def _tc_stage1(xpad, w1, dacc, n, npad):
    d_in = xpad.shape[1]
    d_hid = w1.shape[1]

    def body(x_ref, w_ref, d_ref, y_ref):
        dis = _dis_block(d_ref, n)
        xw = jnp.dot(x_ref[...], w_ref[...],
                     preferred_element_type=jnp.float32,
                     precision=lax.Precision.HIGHEST)
        y_ref[...] = xw * dis

    return pl.pallas_call(
        body,
        grid=(npad // _RB,),
        in_specs=[
            pl.BlockSpec((_RB, d_in), lambda i: (i, 0)),
            _full((d_in, d_hid)),
            _dspec(),
        ],
        out_specs=pl.BlockSpec((_RB, d_hid), lambda i: (i, 0)),
        out_shape=jax.ShapeDtypeStruct((npad, d_hid), jnp.float32),
    )(xpad, w1, dacc)


def _tc_stage2(dacc, a1, y1, b1, w2a, w2b, xroot, n, npad):
    d_hid = y1.shape[1]
    d_out = w2a.shape[1]

    def body(d_ref, a_ref, y_ref, b_ref, wa_ref, wb_ref, xr_ref,
             y2_ref, c1_ref):
        dis = _dis_block(d_ref, n)
        s = a_ref[0, :, :] + a_ref[1, :, :] + y_ref[...]
        conv1 = dis * s + b_ref[...]
        h = jnp.maximum(conv1, 0.0)
        crow = jnp.dot(jnp.maximum(xr_ref[...], 0.0), wb_ref[...],
                       preferred_element_type=jnp.float32,
                       precision=lax.Precision.HIGHEST)
        xw2 = jnp.dot(h, wa_ref[...],
                      preferred_element_type=jnp.float32,
                      precision=lax.Precision.HIGHEST) + crow
        y2_ref[...] = xw2 * dis
        c1_ref[...] = conv1

    return pl.pallas_call(
        body,
        grid=(npad // _RB,),
        in_specs=[
            _dspec(),
            pl.BlockSpec((2, _RB, d_hid), lambda i: (0, i, 0)),
            pl.BlockSpec((_RB, d_hid), lambda i: (i, 0)),
            _full((1, d_hid)),
            _full((d_hid, d_out)),
            _full((d_hid, d_out)),
            _full((1, d_hid)),
        ],
        out_specs=(
            pl.BlockSpec((_RB, d_out), lambda i: (i, 0)),
            pl.BlockSpec((_RB, d_hid), lambda i: (i, 0)),
        ),
        out_shape=(
            jax.ShapeDtypeStruct((npad, d_out), jnp.float32),
            jax.ShapeDtypeStruct((npad, d_hid), jnp.float32),
        ),
    )(dacc, a1, y1, b1, w2a, w2b, xroot)


def _tc_stage3(dacc, a2, y2, b2, c1root, n, npad):
    d_out = y2.shape[1]
    d_hid = c1root.shape[1]
    d_feat = d_hid + d_out

    def body(d_ref, a_ref, y_ref, b_ref, r_ref, o_ref):
        dis = _dis_block(d_ref, n)
        s = a_ref[0, :, :] + a_ref[1, :, :] + y_ref[...]
        conv2 = dis * s + b_ref[...]
        r2 = jnp.maximum(conv2, 0.0)
        f = jnp.concatenate(
            [jnp.broadcast_to(r_ref[...], (_RB, d_hid)), r2], axis=1)
        o_ref[...] = (f[:, 0:d_feat - 2] + f[:, 1:d_feat - 1]
                      + f[:, 2:d_feat]) * (1.0 / 3.0)

    return pl.pallas_call(
        body,
        grid=(npad // _RB,),
        in_specs=[
            _dspec(),
            pl.BlockSpec((2, _RB, d_out), lambda i: (0, i, 0)),
            pl.BlockSpec((_RB, d_out), lambda i: (i, 0)),
            _full((1, d_out)),
            _full((1, d_hid)),
        ],
        out_specs=pl.BlockSpec((_RB, d_feat - 2), lambda i: (i, 0)),
        out_shape=jax.ShapeDtypeStruct((n, d_feat - 2), jnp.float32),
    )(dacc, a2, y2, b2, c1root)


def kernel(x, edge_index, rootIndex, W1, b1, W2, b2):
    n, d_in = x.shape
    d_hid = W1.shape[1]
    e = edge_index.shape[1]
    nw = _NC * _NS

    npad = _ceil_mult(n + 1, _NS * 16)
    assert npad % _RB == 0
    epad = _ceil_mult(e, nw * _CHUNK)
    ncw = epad // (nw * _CHUNK)

    src = edge_index[0]
    dst = edge_index[1]
    pad = epad - e
    fill = jnp.full((pad,), n, jnp.int32)
    srcp = jnp.concatenate([src, fill])
    dstp = jnp.concatenate([dst, fill])
    xpad = jnp.concatenate(
        [x, jnp.zeros((npad - n, d_in), x.dtype)], axis=0)

    dacc = _sc_degree(dstp, npad, ncw)
    y1 = _tc_stage1(xpad, W1, dacc, n, npad)
    a1 = _sc_spmm(y1, srcp, dstp, npad, ncw)
    xroot = lax.dynamic_slice_in_dim(x, rootIndex, 1, axis=0)
    y2, conv1 = _tc_stage2(dacc, a1, y1, b1.reshape(1, -1),
                           W2[:d_hid], W2[d_hid:], xroot, n, npad)
    a2 = _sc_spmm(y2, srcp, dstp, npad, ncw)
    c1root = lax.dynamic_slice_in_dim(conv1, rootIndex, 1, axis=0)
    return _tc_stage3(dacc, a2, y2, b2.reshape(1, -1), c1root, n, npad)
